# baseline (speedup 1.0000x reference)
import functools

import jax
import jax.numpy as jnp
from jax import lax
from jax.experimental import pallas as pl
from jax.experimental.pallas import tpu as pltpu
from jax.experimental.pallas import tpu_sc as plsc

NC = 2
NS = 16
NW = NC * NS
CHUNK = 128


def _sc_gather(table, idx3d, b_total, hidden):
    chunks_per_w = idx3d.shape[1]
    b_per_w = chunks_per_w * CHUNK
    mesh = plsc.VectorSubcoreMesh(core_axis_name="c", subcore_axis_name="s")

    @functools.partial(
        pl.kernel,
        mesh=mesh,
        out_type=jax.ShapeDtypeStruct((b_total, hidden), jnp.float32),
        scratch_types=[
            pltpu.VMEM((chunks_per_w, CHUNK), jnp.int32),
            pltpu.VMEM((CHUNK, hidden), jnp.float32),
            pltpu.SemaphoreType.DMA,
        ],
    )
    def gather_k(table_hbm, idx_hbm, out_hbm, idx_v, rows_v, sem):
        wid = lax.axis_index("s") * NC + lax.axis_index("c")
        base = wid * b_per_w
        pltpu.sync_copy(idx_hbm.at[wid], idx_v)

        def body(j, carry):
            pltpu.async_copy(table_hbm.at[idx_v.at[j]], rows_v, sem).wait()
            pltpu.sync_copy(rows_v, out_hbm.at[pl.ds(base + j * CHUNK, CHUNK)])
            return carry

        lax.fori_loop(0, chunks_per_w, body, 0)

    return gather_k(table, idx3d)


def _tc_linear(x, pos_tile, wt, bias, bm):
    n, hidden = x.shape

    def body(x_ref, p_ref, w_ref, b_ref, o_ref):
        o_ref[...] = (
            jnp.dot(x_ref[...] + p_ref[...], w_ref[...],
                    preferred_element_type=jnp.float32)
            + b_ref[...]
        )

    return pl.pallas_call(
        body,
        grid=(n // bm,),
        in_specs=[
            pl.BlockSpec((bm, hidden), lambda i: (i, 0)),
            pl.BlockSpec((bm, hidden), lambda i: (0, 0)),
            pl.BlockSpec((hidden, hidden), lambda i: (0, 0)),
            pl.BlockSpec((1, hidden), lambda i: (0, 0)),
        ],
        out_specs=pl.BlockSpec((bm, hidden), lambda i: (i, 0)),
        out_shape=jax.ShapeDtypeStruct((n, hidden), jnp.float32),
    )(x, pos_tile, wt, bias)


def kernel(caption, token_embedding, positional_embedding, W, b):
    batch, seq = caption.shape
    vocab, hidden = token_embedding.shape
    b_total = batch * seq

    idx = caption.astype(jnp.int32).reshape(-1)
    n_chunks = b_total // CHUNK
    idx3d = idx.reshape(NW, n_chunks // NW, CHUNK)

    gathered = _sc_gather(token_embedding, idx3d, b_total, hidden)

    caps_per_tile = 32
    bm = caps_per_tile * seq
    pos_tile = jnp.tile(positional_embedding[:seq], (caps_per_tile, 1))
    wt = W.T
    out = _tc_linear(gathered, pos_tile, wt, b.reshape(1, hidden), bm)
    return out.reshape(batch, seq, hidden)


# speedup vs baseline: 1.0766x; 1.0766x over previous
import functools

import jax
import jax.numpy as jnp
from jax import lax
from jax.experimental import pallas as pl
from jax.experimental.pallas import tpu as pltpu
from jax.experimental.pallas import tpu_sc as plsc

NC = 2
NS = 16
NW = NC * NS
CHUNK = 112


def _sc_gather(table, idx3d, b_total, hidden):
    chunks_per_w = idx3d.shape[1]
    assert chunks_per_w % 2 == 0 and chunks_per_w >= 4
    b_per_w = chunks_per_w * CHUNK
    mesh = plsc.VectorSubcoreMesh(core_axis_name="c", subcore_axis_name="s")

    @functools.partial(
        pl.kernel,
        mesh=mesh,
        out_type=jax.ShapeDtypeStruct((b_total, hidden), jnp.float32),
        scratch_types=[
            pltpu.VMEM((chunks_per_w, CHUNK), jnp.int32),
            pltpu.VMEM((CHUNK, hidden), jnp.float32),
            pltpu.VMEM((CHUNK, hidden), jnp.float32),
            pltpu.SemaphoreType.DMA,
            pltpu.SemaphoreType.DMA,
            pltpu.SemaphoreType.DMA,
            pltpu.SemaphoreType.DMA,
        ],
    )
    def gather_k(table_hbm, idx_hbm, out_hbm, idx_v, buf0, buf1, g0, g1, o0, o1):
        wid = lax.axis_index("s") * NC + lax.axis_index("c")
        base = wid * b_per_w
        pltpu.sync_copy(idx_hbm.at[wid], idx_v)

        def gather(j, buf, sem):
            pltpu.async_copy(table_hbm.at[idx_v.at[j]], buf, sem)

        def gather_wait(buf, sem):
            pltpu.make_async_copy(table_hbm.at[idx_v.at[0]], buf, sem).wait()

        def put(j, buf, sem):
            pltpu.async_copy(buf, out_hbm.at[pl.ds(base + j * CHUNK, CHUNK)], sem)

        def put_wait(buf, sem):
            pltpu.make_async_copy(buf, out_hbm.at[pl.ds(base, CHUNK)], sem).wait()

        gather(0, buf0, g0)
        gather(1, buf1, g1)

        def body(i, carry):
            j = 2 * i
            gather_wait(buf0, g0)
            put(j, buf0, o0)
            gather_wait(buf1, g1)
            put(j + 1, buf1, o1)
            put_wait(buf0, o0)
            gather(j + 2, buf0, g0)
            put_wait(buf1, o1)
            gather(j + 3, buf1, g1)
            return carry

        lax.fori_loop(0, chunks_per_w // 2 - 1, body, 0)

        j = chunks_per_w - 2
        gather_wait(buf0, g0)
        put(j, buf0, o0)
        gather_wait(buf1, g1)
        put(j + 1, buf1, o1)
        put_wait(buf0, o0)
        put_wait(buf1, o1)

    return gather_k(table, idx3d)


def _tc_linear(x, pos_tile, wt, bias, bm):
    n, hidden = x.shape

    def body(x_ref, p_ref, w_ref, b_ref, o_ref):
        o_ref[...] = (
            jnp.dot(x_ref[...] + p_ref[...], w_ref[...],
                    preferred_element_type=jnp.float32)
            + b_ref[...]
        )

    return pl.pallas_call(
        body,
        grid=(n // bm,),
        in_specs=[
            pl.BlockSpec((bm, hidden), lambda i: (i, 0)),
            pl.BlockSpec((bm, hidden), lambda i: (0, 0)),
            pl.BlockSpec((hidden, hidden), lambda i: (0, 0)),
            pl.BlockSpec((1, hidden), lambda i: (0, 0)),
        ],
        out_specs=pl.BlockSpec((bm, hidden), lambda i: (i, 0)),
        out_shape=jax.ShapeDtypeStruct((n, hidden), jnp.float32),
    )(x, pos_tile, wt, bias)


def kernel(caption, token_embedding, positional_embedding, W, b):
    batch, seq = caption.shape
    vocab, hidden = token_embedding.shape
    b_total = batch * seq

    idx = caption.astype(jnp.int32).reshape(-1)
    n_chunks = b_total // CHUNK
    idx3d = idx.reshape(NW, n_chunks // NW, CHUNK)

    gathered = _sc_gather(token_embedding, idx3d, b_total, hidden)

    caps_per_tile = 32
    bm = caps_per_tile * seq
    pos_tile = jnp.tile(positional_embedding[:seq], (caps_per_tile, 1))
    wt = W.T
    out = _tc_linear(gathered, pos_tile, wt, b.reshape(1, hidden), bm)
    return out.reshape(batch, seq, hidden)


# speedup vs baseline: 1.8832x; 1.7492x over previous
import functools

import jax
import jax.numpy as jnp
from jax import lax
from jax.experimental import pallas as pl
from jax.experimental.pallas import tpu as pltpu
from jax.experimental.pallas import tpu_sc as plsc

NC = 2
NS = 16
NW = NC * NS
CHUNK = 112


def _sc_gather(table, idx3d, b_total, hidden):
    chunks_per_w = idx3d.shape[1]
    assert chunks_per_w % 2 == 0 and chunks_per_w >= 4
    b_per_w = chunks_per_w * CHUNK
    mesh = plsc.VectorSubcoreMesh(core_axis_name="c", subcore_axis_name="s")

    @functools.partial(
        pl.kernel,
        mesh=mesh,
        out_type=jax.ShapeDtypeStruct((b_total, hidden), jnp.float32),
        scratch_types=[
            pltpu.VMEM((chunks_per_w, CHUNK), jnp.int32),
            pltpu.VMEM((CHUNK, hidden), jnp.float32),
            pltpu.VMEM((CHUNK, hidden), jnp.float32),
            pltpu.SemaphoreType.DMA,
            pltpu.SemaphoreType.DMA,
            pltpu.SemaphoreType.DMA,
            pltpu.SemaphoreType.DMA,
        ],
    )
    def gather_k(table_hbm, idx_hbm, out_hbm, idx_v, buf0, buf1, g0, g1, o0, o1):
        wid = lax.axis_index("s") * NC + lax.axis_index("c")
        base = wid * b_per_w
        pltpu.sync_copy(idx_hbm.at[wid], idx_v)

        def gather(j, buf, sem):
            pltpu.async_copy(table_hbm.at[idx_v.at[j]], buf, sem)

        def gather_wait(buf, sem):
            pltpu.make_async_copy(table_hbm.at[idx_v.at[0]], buf, sem).wait()

        def put(j, buf, sem):
            pltpu.async_copy(buf, out_hbm.at[pl.ds(base + j * CHUNK, CHUNK)], sem)

        def put_wait(buf, sem):
            pltpu.make_async_copy(buf, out_hbm.at[pl.ds(base, CHUNK)], sem).wait()

        gather(0, buf0, g0)
        gather(1, buf1, g1)

        def body(i, carry):
            j = 2 * i
            gather_wait(buf0, g0)
            put(j, buf0, o0)
            gather_wait(buf1, g1)
            put(j + 1, buf1, o1)
            put_wait(buf0, o0)
            gather(j + 2, buf0, g0)
            put_wait(buf1, o1)
            gather(j + 3, buf1, g1)
            return carry

        lax.fori_loop(0, chunks_per_w // 2 - 1, body, 0)

        j = chunks_per_w - 2
        gather_wait(buf0, g0)
        put(j, buf0, o0)
        gather_wait(buf1, g1)
        put(j + 1, buf1, o1)
        put_wait(buf0, o0)
        put_wait(buf1, o1)

    return gather_k(table, idx3d)


def _tc_linear(x, pos, wt, bias, bm, batch):
    n, hidden = x.shape
    blocks_per_pos = batch // bm

    def body(x_ref, p_ref, w_ref, b_ref, o_ref):
        o_ref[...] = (
            jnp.dot(x_ref[...] + p_ref[0], w_ref[...],
                    preferred_element_type=jnp.float32)
            + b_ref[...]
        )

    return pl.pallas_call(
        body,
        grid=(n // bm,),
        in_specs=[
            pl.BlockSpec((bm, hidden), lambda i: (i, 0)),
            pl.BlockSpec((1, 1, hidden), lambda i: (i // blocks_per_pos, 0, 0)),
            pl.BlockSpec((hidden, hidden), lambda i: (0, 0)),
            pl.BlockSpec((1, hidden), lambda i: (0, 0)),
        ],
        out_specs=pl.BlockSpec((bm, hidden), lambda i: (i, 0)),
        out_shape=jax.ShapeDtypeStruct((n, hidden), jnp.float32),
    )(x, pos.reshape(-1, 1, hidden), wt, bias)


def kernel(caption, token_embedding, positional_embedding, W, b):
    batch, seq = caption.shape
    vocab, hidden = token_embedding.shape
    b_total = batch * seq

    idx = caption.astype(jnp.int32).T.reshape(-1)
    n_chunks = b_total // CHUNK
    idx3d = idx.reshape(NW, n_chunks // NW, CHUNK)

    gathered = _sc_gather(token_embedding, idx3d, b_total, hidden)

    bm = 2048
    out = _tc_linear(gathered, positional_embedding[:seq], W.T,
                     b.reshape(1, hidden), bm, batch)
    return out.reshape(seq, batch, hidden).transpose(1, 0, 2)


# speedup vs baseline: 2.1610x; 1.1475x over previous
import functools

import jax
import jax.numpy as jnp
from jax import lax
from jax.experimental import pallas as pl
from jax.experimental.pallas import tpu as pltpu
from jax.experimental.pallas import tpu_sc as plsc

NC = 2
NS = 16
NW = NC * NS
CHUNK = 112
NSLICE = 11
BM = 2048


def _sc_gather(table, idx3d, hidden):
    chunks_per_w = idx3d.shape[1]
    assert chunks_per_w % 2 == 0 and chunks_per_w >= 4
    b_per_w = chunks_per_w * CHUNK
    rows = NW * b_per_w
    mesh = plsc.VectorSubcoreMesh(core_axis_name="c", subcore_axis_name="s")

    @functools.partial(
        pl.kernel,
        mesh=mesh,
        out_type=jax.ShapeDtypeStruct((rows, hidden), jnp.float32),
        scratch_types=[
            pltpu.VMEM((chunks_per_w, CHUNK), jnp.int32),
            pltpu.VMEM((CHUNK, hidden), jnp.float32),
            pltpu.VMEM((CHUNK, hidden), jnp.float32),
            pltpu.SemaphoreType.DMA,
            pltpu.SemaphoreType.DMA,
            pltpu.SemaphoreType.DMA,
            pltpu.SemaphoreType.DMA,
        ],
    )
    def gather_k(table_hbm, idx_hbm, out_hbm, idx_v, buf0, buf1, g0, g1, o0, o1):
        wid = lax.axis_index("s") * NC + lax.axis_index("c")
        base = wid * b_per_w
        pltpu.sync_copy(idx_hbm.at[wid], idx_v)

        def gather(j, buf, sem):
            pltpu.async_copy(table_hbm.at[idx_v.at[j]], buf, sem)

        def gather_wait(buf, sem):
            pltpu.make_async_copy(table_hbm.at[idx_v.at[0]], buf, sem).wait()

        def put(j, buf, sem):
            pltpu.async_copy(buf, out_hbm.at[pl.ds(base + j * CHUNK, CHUNK)], sem)

        def put_wait(buf, sem):
            pltpu.make_async_copy(buf, out_hbm.at[pl.ds(base, CHUNK)], sem).wait()

        gather(0, buf0, g0)
        gather(1, buf1, g1)

        def body(i, carry):
            j = 2 * i
            gather_wait(buf0, g0)
            put(j, buf0, o0)
            gather_wait(buf1, g1)
            put(j + 1, buf1, o1)
            put_wait(buf0, o0)
            gather(j + 2, buf0, g0)
            put_wait(buf1, o1)
            gather(j + 3, buf1, g1)
            return carry

        lax.fori_loop(0, chunks_per_w // 2 - 1, body, 0)

        j = chunks_per_w - 2
        gather_wait(buf0, g0)
        put(j, buf0, o0)
        gather_wait(buf1, g1)
        put(j + 1, buf1, o1)
        put_wait(buf0, o0)
        put_wait(buf1, o1)

    return gather_k(table, idx3d)


def _tc_linear_slice(x_slice, pos3, wt, bias, out_prev, row0, batch):
    n, hidden = x_slice.shape
    n_total = pos3.shape[0] * batch
    nb = n // BM
    block0 = row0 // BM
    bpp = batch // BM

    def body(x_ref, p_ref, w_ref, b_ref, *rest):
        o_ref = rest[-1]
        o_ref[...] = (
            jnp.dot(x_ref[...] + p_ref[0], w_ref[...],
                    preferred_element_type=jnp.float32)
            + b_ref[...]
        )

    in_specs = [
        pl.BlockSpec((BM, hidden), lambda i: (i, 0)),
        pl.BlockSpec((1, 1, hidden), lambda i: (block0 // bpp + i // bpp, 0, 0)),
        pl.BlockSpec((hidden, hidden), lambda i: (0, 0)),
        pl.BlockSpec((1, hidden), lambda i: (0, 0)),
    ]
    args = [x_slice, pos3, wt, bias]
    io_aliases = {}
    if out_prev is not None:
        in_specs.append(pl.BlockSpec(memory_space=pl.ANY))
        args.append(out_prev)
        io_aliases = {4: 0}

    return pl.pallas_call(
        body,
        grid=(nb,),
        in_specs=in_specs,
        out_specs=pl.BlockSpec((BM, hidden), lambda i: (block0 + i, 0)),
        out_shape=jax.ShapeDtypeStruct((n_total, hidden), jnp.float32),
        input_output_aliases=io_aliases,
    )(*args)


def kernel(caption, token_embedding, positional_embedding, W, b):
    batch, seq = caption.shape
    vocab, hidden = token_embedding.shape
    b_total = batch * seq

    idx = caption.astype(jnp.int32).T.reshape(-1)
    n_chunks = b_total // CHUNK
    chunks_per_w = n_chunks // NW
    cw_slice = chunks_per_w // NSLICE
    rows_slice = NW * cw_slice * CHUNK
    idx4d = idx.reshape(NSLICE, NW, cw_slice, CHUNK)

    pos3 = positional_embedding[:seq].reshape(seq, 1, hidden)
    wt = W.T
    bias = b.reshape(1, hidden)

    out = None
    for k in range(NSLICE):
        idx_k = idx4d[k]
        gathered_k = _sc_gather(token_embedding, idx_k, hidden)
        out = _tc_linear_slice(gathered_k, pos3, wt, bias, out,
                               k * rows_slice, batch)

    return out.reshape(seq, batch, hidden).transpose(1, 0, 2)


# speedup vs baseline: 2.2172x; 1.0260x over previous
import functools

import jax
import jax.numpy as jnp
from jax import lax
from jax.experimental import pallas as pl
from jax.experimental.pallas import tpu as pltpu
from jax.experimental.pallas import tpu_sc as plsc

NC = 2
NS = 16
NW = NC * NS
CHUNK = 112
NSLICE = 11
BM = 4096


def _sc_gather(table, idx3d, hidden):
    chunks_per_w = idx3d.shape[1]
    assert chunks_per_w % 2 == 0 and chunks_per_w >= 4
    b_per_w = chunks_per_w * CHUNK
    rows = NW * b_per_w
    mesh = plsc.VectorSubcoreMesh(core_axis_name="c", subcore_axis_name="s")

    @functools.partial(
        pl.kernel,
        mesh=mesh,
        out_type=jax.ShapeDtypeStruct((rows, hidden), jnp.float32),
        scratch_types=[
            pltpu.VMEM((chunks_per_w, CHUNK), jnp.int32),
            pltpu.VMEM((CHUNK, hidden), jnp.float32),
            pltpu.VMEM((CHUNK, hidden), jnp.float32),
            pltpu.SemaphoreType.DMA,
            pltpu.SemaphoreType.DMA,
            pltpu.SemaphoreType.DMA,
            pltpu.SemaphoreType.DMA,
        ],
    )
    def gather_k(table_hbm, idx_hbm, out_hbm, idx_v, buf0, buf1, g0, g1, o0, o1):
        wid = lax.axis_index("s") * NC + lax.axis_index("c")
        base = wid * b_per_w
        pltpu.sync_copy(idx_hbm.at[wid], idx_v)

        def gather(j, buf, sem):
            pltpu.async_copy(table_hbm.at[idx_v.at[j]], buf, sem)

        def gather_wait(buf, sem):
            pltpu.make_async_copy(table_hbm.at[idx_v.at[0]], buf, sem).wait()

        def put(j, buf, sem):
            pltpu.async_copy(buf, out_hbm.at[pl.ds(base + j * CHUNK, CHUNK)], sem)

        def put_wait(buf, sem):
            pltpu.make_async_copy(buf, out_hbm.at[pl.ds(base, CHUNK)], sem).wait()

        gather(0, buf0, g0)
        gather(1, buf1, g1)

        def body(i, carry):
            j = 2 * i
            gather_wait(buf0, g0)
            put(j, buf0, o0)
            gather_wait(buf1, g1)
            put(j + 1, buf1, o1)
            put_wait(buf0, o0)
            gather(j + 2, buf0, g0)
            put_wait(buf1, o1)
            gather(j + 3, buf1, g1)
            return carry

        lax.fori_loop(0, chunks_per_w // 2 - 1, body, 0)

        j = chunks_per_w - 2
        gather_wait(buf0, g0)
        put(j, buf0, o0)
        gather_wait(buf1, g1)
        put(j + 1, buf1, o1)
        put_wait(buf0, o0)
        put_wait(buf1, o1)

    return gather_k(table, idx3d)


def _tc_linear_slice(x_slice, pos3, wt, bias, out_prev, row0, batch):
    n, hidden = x_slice.shape
    n_total = pos3.shape[0] * batch
    nb = n // BM
    block0 = row0 // BM
    bpp = batch // BM

    def body(x_ref, p_ref, w_ref, b_ref, *rest):
        o_ref = rest[-1]
        o_ref[...] = (
            jnp.dot(x_ref[...] + p_ref[0], w_ref[...],
                    preferred_element_type=jnp.float32)
            + b_ref[...]
        )

    in_specs = [
        pl.BlockSpec((BM, hidden), lambda i: (i, 0)),
        pl.BlockSpec((1, 1, hidden), lambda i: (block0 // bpp + i // bpp, 0, 0)),
        pl.BlockSpec((hidden, hidden), lambda i: (0, 0)),
        pl.BlockSpec((1, hidden), lambda i: (0, 0)),
    ]
    args = [x_slice, pos3, wt, bias]
    io_aliases = {}
    if out_prev is not None:
        in_specs.append(pl.BlockSpec(memory_space=pl.ANY))
        args.append(out_prev)
        io_aliases = {4: 0}

    return pl.pallas_call(
        body,
        grid=(nb,),
        in_specs=in_specs,
        out_specs=pl.BlockSpec((BM, hidden), lambda i: (block0 + i, 0)),
        out_shape=jax.ShapeDtypeStruct((n_total, hidden), jnp.float32),
        input_output_aliases=io_aliases,
    )(*args)


def kernel(caption, token_embedding, positional_embedding, W, b):
    batch, seq = caption.shape
    vocab, hidden = token_embedding.shape
    b_total = batch * seq

    idx = caption.astype(jnp.int32).T.reshape(-1)
    n_chunks = b_total // CHUNK
    chunks_per_w = n_chunks // NW
    cw_slice = chunks_per_w // NSLICE
    rows_slice = NW * cw_slice * CHUNK
    idx4d = idx.reshape(NSLICE, NW, cw_slice, CHUNK)

    pos3 = positional_embedding[:seq].reshape(seq, 1, hidden)
    wt = W.T
    bias = b.reshape(1, hidden)

    out = None
    for k in range(NSLICE):
        idx_k = idx4d[k]
        gathered_k = _sc_gather(token_embedding, idx_k, hidden)
        out = _tc_linear_slice(gathered_k, pos3, wt, bias, out,
                               k * rows_slice, batch)

    return out.reshape(seq, batch, hidden).transpose(1, 0, 2)
